# baseline (speedup 1.0000x reference)
import jax
import jax.numpy as jnp
from jax.experimental import pallas as pl
from jax.experimental.pallas import tpu as pltpu

_TI = 2000
_W0 = 1536
_WT = 784
_NK = 7
_RM = 3


def _gc_body(xt_ref, adj_ref, w_ref, out_ref, main_ref, tail_ref,
             sm_ref, st_ref):
    i = pl.program_id(0)
    k = pl.num_programs(0)

    def main_copy(step, slot):
        r = step // _NK
        kk = step % _NK
        return pltpu.make_async_copy(
            adj_ref.at[pl.ds(r * _TI, _TI), pl.ds(kk * _W0, _W0)],
            main_ref.at[slot],
            sm_ref.at[slot],
        )

    def tail_copy(step):
        r = step // _NK
        return pltpu.make_async_copy(
            adj_ref.at[pl.ds(r * _TI, _TI), pl.ds((_NK - 1) * _W0, _WT)],
            tail_ref,
            st_ref,
        )

    def start_copy(step):
        kk = step % _NK
        m = step - step // _NK

        @pl.when(kk < _NK - 1)
        def _():
            main_copy(step, jax.lax.rem(m, _RM)).start()

        @pl.when(kk == _NK - 1)
        def _():
            tail_copy(step).start()

    @pl.when(i == 0)
    def _init():
        out_ref[...] = jnp.zeros_like(out_ref)
        start_copy(0)
        start_copy(1)
        start_copy(2)

    kk = jax.lax.rem(i, _NK)
    r = jax.lax.div(i, _NK)
    m = i - r
    w_blk = w_ref[pl.ds(r * _TI, _TI), :]

    def accumulate(a_pnl):
        out_ref[...] += jax.lax.dot_general(
            a_pnl, w_blk,
            (((0,), (0,)), ((), ())),
            preferred_element_type=jnp.float32,
        )

    @pl.when(kk < _NK - 1)
    def _main():
        slot = jax.lax.rem(m, _RM)
        main_copy(i, slot).wait()
        a_pnl = jax.lax.dot_general(
            main_ref[slot], xt_ref[pl.ds(kk * _W0, _W0), :],
            (((1,), (0,)), ((), ())),
            preferred_element_type=jnp.float32,
        )
        accumulate(a_pnl)

    @pl.when(kk == _NK - 1)
    def _tail():
        tail_copy(i).wait()
        a_pnl = jax.lax.dot_general(
            tail_ref[...], xt_ref[pl.ds((_NK - 1) * _W0, _WT), :],
            (((1,), (0,)), ((), ())),
            preferred_element_type=jnp.float32,
        )
        accumulate(a_pnl)

    nxt = i + _RM

    @pl.when(nxt < k)
    def _prefetch():
        start_copy(nxt)


def kernel(x, adj, weight):
    d, n = x.shape
    f = weight.shape[1]
    xt = x.T
    grid = ((n // _TI) * _NK,)
    return pl.pallas_call(
        _gc_body,
        grid=grid,
        in_specs=[
            pl.BlockSpec((n, d), lambda i: (0, 0)),
            pl.BlockSpec(memory_space=pl.ANY),
            pl.BlockSpec((n, f), lambda i: (0, 0)),
        ],
        out_specs=pl.BlockSpec((d, f), lambda i: (0, 0)),
        out_shape=jax.ShapeDtypeStruct((d, f), jnp.float32),
        scratch_shapes=[
            pltpu.VMEM((_RM, _TI, _W0), jnp.float32),
            pltpu.VMEM((_TI, _WT), jnp.float32),
            pltpu.SemaphoreType.DMA((_RM,)),
            pltpu.SemaphoreType.DMA,
        ],
        compiler_params=pltpu.CompilerParams(
            dimension_semantics=("arbitrary",),
        ),
    )(xt, adj, weight)


# speedup vs baseline: 1.0123x; 1.0123x over previous
import jax
import jax.numpy as jnp
from jax.experimental import pallas as pl
from jax.experimental.pallas import tpu as pltpu

_TI = 1000
_W0 = 1280
_WT = 1040
_NK = 8
_RM = 6


def _gc_body(xt_ref, adj_ref, w_ref, out_ref, main_ref, tail_ref,
             sm_ref, st_ref):
    i = pl.program_id(0)
    k = pl.num_programs(0)

    def main_copy(step, slot):
        r = step // _NK
        kk = step % _NK
        return pltpu.make_async_copy(
            adj_ref.at[pl.ds(r * _TI, _TI), pl.ds(kk * _W0, _W0)],
            main_ref.at[slot],
            sm_ref.at[slot],
        )

    def tail_copy(step):
        r = step // _NK
        return pltpu.make_async_copy(
            adj_ref.at[pl.ds(r * _TI, _TI), pl.ds(7 * _W0, _WT)],
            tail_ref,
            st_ref,
        )

    def start_copy(step):
        kk = step % _NK
        m = step - step // _NK

        @pl.when(kk < _NK - 1)
        def _():
            main_copy(step, jax.lax.rem(m, _RM)).start()

        @pl.when(kk == _NK - 1)
        def _():
            tail_copy(step).start()

    @pl.when(i == 0)
    def _init():
        out_ref[...] = jnp.zeros_like(out_ref)
        for s in range(_RM):
            start_copy(s)

    kk = jax.lax.rem(i, _NK)
    r = jax.lax.div(i, _NK)
    m = i - r
    w_blk = w_ref[pl.ds(r * _TI, _TI), :]

    def accumulate(a_pnl):
        out_ref[...] += jax.lax.dot_general(
            a_pnl, w_blk,
            (((0,), (0,)), ((), ())),
            preferred_element_type=jnp.float32,
        )

    @pl.when(kk < _NK - 1)
    def _main():
        slot = jax.lax.rem(m, _RM)
        main_copy(i, slot).wait()
        a_pnl = jax.lax.dot_general(
            main_ref[slot], xt_ref[pl.ds(kk * _W0, _W0), :],
            (((1,), (0,)), ((), ())),
            preferred_element_type=jnp.float32,
        )
        accumulate(a_pnl)

    @pl.when(kk == _NK - 1)
    def _tail():
        tail_copy(i).wait()
        a_pnl = jax.lax.dot_general(
            tail_ref[...], xt_ref[pl.ds(7 * _W0, _WT), :],
            (((1,), (0,)), ((), ())),
            preferred_element_type=jnp.float32,
        )
        accumulate(a_pnl)

    nxt = i + _RM

    @pl.when(nxt < k)
    def _prefetch():
        start_copy(nxt)


def kernel(x, adj, weight):
    d, n = x.shape
    f = weight.shape[1]
    xt = x.T
    grid = ((n // _TI) * _NK,)
    return pl.pallas_call(
        _gc_body,
        grid=grid,
        in_specs=[
            pl.BlockSpec((n, d), lambda i: (0, 0)),
            pl.BlockSpec(memory_space=pl.ANY),
            pl.BlockSpec((n, f), lambda i: (0, 0)),
        ],
        out_specs=pl.BlockSpec((d, f), lambda i: (0, 0)),
        out_shape=jax.ShapeDtypeStruct((d, f), jnp.float32),
        scratch_shapes=[
            pltpu.VMEM((_RM, _TI, _W0), jnp.float32),
            pltpu.VMEM((_TI, _WT), jnp.float32),
            pltpu.SemaphoreType.DMA((_RM,)),
            pltpu.SemaphoreType.DMA,
        ],
        compiler_params=pltpu.CompilerParams(
            dimension_semantics=("arbitrary",),
        ),
    )(xt, adj, weight)


# speedup vs baseline: 1.0239x; 1.0115x over previous
import jax
import jax.numpy as jnp
from jax.experimental import pallas as pl
from jax.experimental.pallas import tpu as pltpu

_TI = 2000
_W0 = 1280
_WT = 1040
_NK = 8
_RM = 3


def _gc_body(xt_ref, adj_ref, w_ref, out_ref, main_ref, tail_ref, acc_ref,
             sm_ref, st_ref):
    i = pl.program_id(0)
    k = pl.num_programs(0)

    def main_copy(step, slot):
        r = step // _NK
        kk = step % _NK
        return pltpu.make_async_copy(
            adj_ref.at[pl.ds(r * _TI, _TI), pl.ds(kk * _W0, _W0)],
            main_ref.at[slot],
            sm_ref.at[slot],
        )

    def tail_copy(step):
        r = step // _NK
        return pltpu.make_async_copy(
            adj_ref.at[pl.ds(r * _TI, _TI), pl.ds((_NK - 1) * _W0, _WT)],
            tail_ref,
            st_ref,
        )

    def start_copy(step):
        kk = step % _NK
        m = step - step // _NK

        @pl.when(kk < _NK - 1)
        def _():
            main_copy(step, jax.lax.rem(m, _RM)).start()

        @pl.when(kk == _NK - 1)
        def _():
            tail_copy(step).start()

    @pl.when(i == 0)
    def _init():
        out_ref[...] = jnp.zeros_like(out_ref)
        start_copy(0)
        start_copy(1)
        start_copy(2)

    kk = jax.lax.rem(i, _NK)
    r = jax.lax.div(i, _NK)
    m = i - r

    @pl.when(kk < _NK - 1)
    def _main():
        slot = jax.lax.rem(m, _RM)
        main_copy(i, slot).wait()
        a_pnl = jax.lax.dot_general(
            main_ref[slot], xt_ref[pl.ds(kk * _W0, _W0), :],
            (((1,), (0,)), ((), ())),
            preferred_element_type=jnp.float32,
        )

        @pl.when(kk == 0)
        def _():
            acc_ref[...] = a_pnl

        @pl.when(kk > 0)
        def _():
            acc_ref[...] += a_pnl

    @pl.when(kk == _NK - 1)
    def _tail():
        tail_copy(i).wait()
        a_row = acc_ref[...] + jax.lax.dot_general(
            tail_ref[...], xt_ref[pl.ds((_NK - 1) * _W0, _WT), :],
            (((1,), (0,)), ((), ())),
            preferred_element_type=jnp.float32,
        )
        out_ref[...] += jax.lax.dot_general(
            a_row, w_ref[pl.ds(r * _TI, _TI), :],
            (((0,), (0,)), ((), ())),
            preferred_element_type=jnp.float32,
        )

    nxt = i + _RM

    @pl.when(nxt < k)
    def _prefetch():
        start_copy(nxt)


def kernel(x, adj, weight):
    d, n = x.shape
    f = weight.shape[1]
    xt = x.T
    grid = ((n // _TI) * _NK,)
    return pl.pallas_call(
        _gc_body,
        grid=grid,
        in_specs=[
            pl.BlockSpec((n, d), lambda i: (0, 0)),
            pl.BlockSpec(memory_space=pl.ANY),
            pl.BlockSpec((n, f), lambda i: (0, 0)),
        ],
        out_specs=pl.BlockSpec((d, f), lambda i: (0, 0)),
        out_shape=jax.ShapeDtypeStruct((d, f), jnp.float32),
        scratch_shapes=[
            pltpu.VMEM((_RM, _TI, _W0), jnp.float32),
            pltpu.VMEM((_TI, _WT), jnp.float32),
            pltpu.VMEM((_TI, 128), jnp.float32),
            pltpu.SemaphoreType.DMA((_RM,)),
            pltpu.SemaphoreType.DMA,
        ],
        compiler_params=pltpu.CompilerParams(
            dimension_semantics=("arbitrary",),
        ),
    )(xt, adj, weight)
